# baseline (speedup 1.0000x reference)
import functools

import jax
import jax.numpy as jnp
from jax import lax
from jax.experimental import pallas as pl
from jax.experimental.pallas import tpu as pltpu
from jax.experimental.pallas import tpu_sc as plsc

_L16 = 16
_NC = 2
_NS = 16
_NW = _NC * _NS
_B = 1024
_NL = 32
_ND = 32
_NE = 32
_BBLK = 64
_NBB = _B // _BBLK
_BC = _BBLK // _L16


def _rsqrt_nr(x):
    i = plsc.bitcast(x, jnp.int32)
    i = jnp.int32(0x5F3759DF) - (i >> 1)
    y = plsc.bitcast(i, jnp.float32)
    for _ in range(3):
        y = y * (jnp.float32(1.5) - jnp.float32(0.5) * x * y * y)
    return y


def _make_sc_kernel():
    mesh = plsc.VectorSubcoreMesh(core_axis_name="c", subcore_axis_name="s")
    f32 = jnp.float32

    @functools.partial(
        pl.kernel,
        mesh=mesh,
        compiler_params=pltpu.CompilerParams(needs_layout_passes=False,
                                             use_tc_tiling_on_sc=False),
        out_type=jax.ShapeDtypeStruct((_NL, _ND, _NE, _B), f32),
        scratch_types=[
            pltpu.VMEM((_ND, _NBB, _BBLK), jnp.int32),
            pltpu.VMEM((_BBLK, _NE), f32),
            pltpu.VMEM((_BBLK, _NE), f32),
            pltpu.VMEM((_ND, _NE, _BBLK), f32),
            pltpu.SemaphoreType.DMA,
            pltpu.SemaphoreType.DMA,
            pltpu.SemaphoreType.DMA,
        ],
    )
    def sc_kernel(idx_hbm, tab_hbm, out_hbm, idx_v, gb0, gb1, tbuf,
                  sg0, sg1, so):
        l = lax.axis_index("s") * _NC + lax.axis_index("c")
        pltpu.sync_copy(idx_hbm.at[l], idx_v)

        iota16 = lax.iota(jnp.int32, _L16)
        lane_sel = [iota16 + jnp.int32(bc * _L16) for bc in range(_BC)]
        col_sel = [jnp.full((_L16,), e, jnp.int32) for e in range(_NE)]
        inv_n = jnp.float32(1.0 / (_ND * _NE))
        zero = jnp.zeros((_L16,), f32)
        gbufs = ((gb0, sg0), (gb1, sg1))

        def bblock(blk, carry):
            @pl.when(blk > 0)
            def _():
                for _d in range(_ND):
                    pltpu.make_async_copy(
                        tbuf.at[0], out_hbm.at[0, 0, :, pl.ds(0, _BBLK)],
                        so).wait()

            for u in (0, 1):
                gb, sg = gbufs[u]
                pltpu.async_copy(tab_hbm.at[idx_v.at[u, blk]], gb, sg)

            def dpair(jd, accs):
                for u in (0, 1):
                    d = jd * 2 + u
                    gb, sg = gbufs[u]
                    pltpu.make_async_copy(
                        tab_hbm.at[pl.ds(0, _BBLK)], gb, sg).wait()
                    new_accs = list(accs)
                    for e in range(_NE):
                        for bc in range(_BC):
                            v = plsc.load_gather(gb, [lane_sel[bc], col_sel[e]])
                            tbuf[d, e, pl.ds(bc * _L16, _L16)] = v
                            new_accs[bc] = new_accs[bc] + v
                            new_accs[_BC + bc] = new_accs[_BC + bc] + v * v
                    accs = tuple(new_accs)
                    @pl.when(d + 2 < _ND)
                    def _():
                        pltpu.async_copy(
                            tab_hbm.at[idx_v.at[d + 2, blk]], gb, sg)
                return accs

            accs = lax.fori_loop(0, _ND // 2, dpair, (zero,) * (2 * _BC))

            scales, shifts = [], []
            for bc in range(_BC):
                mean = accs[bc] * inv_n
                var = jnp.maximum(accs[_BC + bc] * inv_n - mean * mean,
                                  jnp.float32(0.0))
                sc = _rsqrt_nr(var + jnp.float32(1e-5))
                scales.append(sc)
                shifts.append(mean * sc)

            def dnorm(d, carry):
                for e in range(_NE):
                    for bc in range(_BC):
                        v = tbuf[d, e, pl.ds(bc * _L16, _L16)]
                        tbuf[d, e, pl.ds(bc * _L16, _L16)] = (
                            v * scales[bc] - shifts[bc])
                pltpu.async_copy(
                    tbuf.at[d],
                    out_hbm.at[l, d, :, pl.ds(blk * _BBLK, _BBLK)], so)
                return carry

            lax.fori_loop(0, _ND, dnorm, 0)
            return carry

        lax.fori_loop(0, _NBB, bblock, 0)

        for _d in range(_ND):
            pltpu.make_async_copy(
                tbuf.at[0], out_hbm.at[0, 0, :, pl.ds(0, _BBLK)], so).wait()

    return sc_kernel


_sc_kernel = _make_sc_kernel()


def kernel(x, emb_weight, pos_weight):
    tab = emb_weight + pos_weight
    xt = jnp.transpose(x, (1, 2, 0)).reshape(_NL, _ND, _NBB, _BBLK)
    o = _sc_kernel(xt, tab)
    return jnp.transpose(o, (3, 0, 1, 2))


# speedup vs baseline: 1.2492x; 1.2492x over previous
import functools

import jax
import jax.numpy as jnp
from jax import lax
from jax.experimental import pallas as pl
from jax.experimental.pallas import tpu as pltpu
from jax.experimental.pallas import tpu_sc as plsc

_EMBED = 32
_L = 16
_NC = 2
_NS = 16
_NW = _NC * _NS
_CHUNK = 128
_GROUP = 32
_GROUPS_PER_CHUNK = _CHUNK // _GROUP
_N_ROWS = 1024 * 32 * 32
_N_CHUNKS = _N_ROWS // _CHUNK
_CHUNKS_PER_W = _N_CHUNKS // _NW
_UNROLL = 4


def _lane_sum(v):
    lanes = lax.iota(jnp.int32, _L)
    dnums = lax.GatherDimensionNumbers(
        offset_dims=(), collapsed_slice_dims=(0,), start_index_map=(0,))
    for sh in (8, 4, 2, 1):
        perm = lax.gather(v, (lanes ^ sh)[:, None], dnums, slice_sizes=(1,),
                          mode=lax.GatherScatterMode.PROMISE_IN_BOUNDS)
        v = v + perm
    return v


def _rsqrt_nr(x):
    i = plsc.bitcast(x, jnp.int32)
    i = jnp.int32(0x5F3759DF) - (i >> 1)
    y = plsc.bitcast(i, jnp.float32)
    for _ in range(3):
        y = y * (jnp.float32(1.5) - jnp.float32(0.5) * x * y * y)
    return y


def _compute_chunk(ea, ob):
    for g in range(_GROUPS_PER_CHUNK):
        g0 = g * _GROUP

        def pass1(r, acc):
            s, ss = acc
            for u in range(_UNROLL):
                row = g0 + r * _UNROLL + u
                y0 = ea[row, pl.ds(0, _L)]
                y1 = ea[row, pl.ds(_L, _L)]
                s = s + (y0 + y1)
                ss = ss + (y0 * y0 + y1 * y1)
            return s, ss

        zero = jnp.zeros((_L,), jnp.float32)
        s, ss = lax.fori_loop(0, _GROUP // _UNROLL, pass1, (zero, zero))
        inv_n = jnp.float32(1.0 / (_GROUP * _EMBED))
        mean_v = _lane_sum(s) * inv_n
        var_v = jnp.maximum(_lane_sum(ss) * inv_n - mean_v * mean_v,
                            jnp.float32(0.0))
        scale_v = _rsqrt_nr(var_v + jnp.float32(1e-5))
        shift_v = mean_v * scale_v

        def pass2(r, carry):
            for u in range(_UNROLL):
                row = g0 + r * _UNROLL + u
                ob[row, pl.ds(0, _L)] = ea[row, pl.ds(0, _L)] * scale_v - shift_v
                ob[row, pl.ds(_L, _L)] = ea[row, pl.ds(_L, _L)] * scale_v - shift_v
            return carry

        lax.fori_loop(0, _GROUP // _UNROLL, pass2, 0)


def _make_sc_kernel():
    mesh = plsc.VectorSubcoreMesh(core_axis_name="c", subcore_axis_name="s")
    f32 = jnp.float32

    @functools.partial(
        pl.kernel,
        mesh=mesh,
        compiler_params=pltpu.CompilerParams(needs_layout_passes=False,
                                             use_tc_tiling_on_sc=False),
        out_type=jax.ShapeDtypeStruct((_N_ROWS, _EMBED), f32),
        scratch_types=[
            pltpu.VMEM((32, 32, 32), jnp.int32),
            pltpu.VMEM((_CHUNKS_PER_W, _CHUNK), jnp.int32),
            pltpu.VMEM((_CHUNK, _EMBED), f32),
            pltpu.VMEM((_CHUNK, _EMBED), f32),
            pltpu.VMEM((_CHUNK, _EMBED), f32),
            pltpu.VMEM((_CHUNK, _EMBED), f32),
            pltpu.SemaphoreType.DMA,
            pltpu.SemaphoreType.DMA,
            pltpu.SemaphoreType.DMA,
            pltpu.SemaphoreType.DMA,
        ],
    )
    def sc_kernel(idx_hbm, tab_hbm, out_hbm, idx_n, idx_v,
                  ea0, ob0, ea1, ob1, sg0, sg1, so0, so1):
        wid = lax.axis_index("s") * _NC + lax.axis_index("c")
        chunk0 = wid * _CHUNKS_PER_W
        bufs = ((ea0, ob0, sg0, so0), (ea1, ob1, sg1, so1))

        pltpu.sync_copy(idx_hbm.at[:, :, pl.ds(wid * 32, 32)], idx_n)
        iota16 = lax.iota(jnp.int32, _L)

        def stage(v, carry):
            il = jnp.full((_L,), (v >> 1) & 31, jnp.int32)
            id_ = iota16 + (v & 1) * _L
            ib = jnp.full((_L,), v >> 6, jnp.int32)
            vals = plsc.load_gather(idx_n, [il, id_, ib])
            idx_v[v >> 3, pl.ds((v & 7) * _L, _L)] = vals
            return carry

        lax.fori_loop(0, 2048, stage, 0)

        for b in (0, 1):
            ea, _, sg, _ = bufs[b]
            pltpu.async_copy(tab_hbm.at[idx_v.at[b]], ea, sg)

        n_iter = _CHUNKS_PER_W // 2

        def body(j, carry):
            for b in (0, 1):
                ea, ob, sg, so = bufs[b]
                c = j * 2 + b
                pltpu.make_async_copy(tab_hbm.at[pl.ds(0, _CHUNK)], ea, sg).wait()

                @pl.when(j > 0)
                def _():
                    pltpu.make_async_copy(
                        ob, out_hbm.at[pl.ds(0, _CHUNK)], so).wait()

                _compute_chunk(ea, ob)

                row0 = (chunk0 + c) * _CHUNK
                pltpu.async_copy(ob, out_hbm.at[pl.ds(row0, _CHUNK)], so)

                @pl.when(j < n_iter - 1)
                def _():
                    pltpu.async_copy(tab_hbm.at[idx_v.at[c + 2]], ea, sg)
            return carry

        lax.fori_loop(0, n_iter, body, 0)

        for b in (0, 1):
            _, ob, _, so = bufs[b]
            pltpu.make_async_copy(ob, out_hbm.at[pl.ds(0, _CHUNK)], so).wait()

    return sc_kernel


_sc_kernel = _make_sc_kernel()


def kernel(x, emb_weight, pos_weight):
    b, l, d = x.shape
    e = emb_weight.shape[1]
    tab = emb_weight + pos_weight
    xt = jnp.transpose(x, (1, 2, 0))
    out = _sc_kernel(xt, tab)
    return out.reshape(b, l, d, e)


# speedup vs baseline: 1.2497x; 1.0004x over previous
import functools

import jax
import jax.numpy as jnp
from jax import lax
from jax.experimental import pallas as pl
from jax.experimental.pallas import tpu as pltpu
from jax.experimental.pallas import tpu_sc as plsc

_EMBED = 32
_L = 16
_NC = 2
_NS = 16
_NW = _NC * _NS
_CHUNK = 128
_GROUP = 32
_GROUPS_PER_CHUNK = _CHUNK // _GROUP
_N_ROWS = 1024 * 32 * 32
_N_CHUNKS = _N_ROWS // _CHUNK
_CHUNKS_PER_W = _N_CHUNKS // _NW
_UNROLL = 4


def _lane_sum(v):
    lanes = lax.iota(jnp.int32, _L)
    dnums = lax.GatherDimensionNumbers(
        offset_dims=(), collapsed_slice_dims=(0,), start_index_map=(0,))
    for sh in (8, 4, 2, 1):
        perm = lax.gather(v, (lanes ^ sh)[:, None], dnums, slice_sizes=(1,),
                          mode=lax.GatherScatterMode.PROMISE_IN_BOUNDS)
        v = v + perm
    return v


def _rsqrt_nr(x):
    i = plsc.bitcast(x, jnp.int32)
    i = jnp.int32(0x5F3759DF) - (i >> 1)
    y = plsc.bitcast(i, jnp.float32)
    for _ in range(3):
        y = y * (jnp.float32(1.5) - jnp.float32(0.5) * x * y * y)
    return y


def _compute_chunk(ea, ob):
    for g in range(_GROUPS_PER_CHUNK):
        g0 = g * _GROUP

        def pass1(r, acc):
            s, ss = acc
            for u in range(_UNROLL):
                row = g0 + r * _UNROLL + u
                y0 = ea[row, pl.ds(0, _L)]
                y1 = ea[row, pl.ds(_L, _L)]
                s = s + (y0 + y1)
                ss = ss + (y0 * y0 + y1 * y1)
            return s, ss

        zero = jnp.zeros((_L,), jnp.float32)
        s, ss = lax.fori_loop(0, _GROUP // _UNROLL, pass1, (zero, zero))
        inv_n = jnp.float32(1.0 / (_GROUP * _EMBED))
        mean_v = _lane_sum(s) * inv_n
        var_v = jnp.maximum(_lane_sum(ss) * inv_n - mean_v * mean_v,
                            jnp.float32(0.0))
        scale_v = _rsqrt_nr(var_v + jnp.float32(1e-5))
        shift_v = mean_v * scale_v

        def pass2(r, carry):
            prow = (g0 >> 2) + r
            for u in range(_UNROLL):
                row = g0 + r * _UNROLL + u
                ob[prow, pl.ds(u * 32, _L)] = (
                    ea[row, pl.ds(0, _L)] * scale_v - shift_v)
                ob[prow, pl.ds(u * 32 + _L, _L)] = (
                    ea[row, pl.ds(_L, _L)] * scale_v - shift_v)
            return carry

        lax.fori_loop(0, _GROUP // _UNROLL, pass2, 0)


def _make_sc_kernel():
    mesh = plsc.VectorSubcoreMesh(core_axis_name="c", subcore_axis_name="s")
    f32 = jnp.float32

    @functools.partial(
        pl.kernel,
        mesh=mesh,
        compiler_params=pltpu.CompilerParams(needs_layout_passes=False,
                                             use_tc_tiling_on_sc=False),
        out_type=jax.ShapeDtypeStruct((_N_ROWS // 4, 4 * _EMBED), f32),
        scratch_types=[
            pltpu.VMEM((32, 32, 32), f32),
            pltpu.VMEM((_CHUNKS_PER_W, _CHUNK), jnp.int32),
            pltpu.VMEM((_CHUNK, _EMBED), f32),
            pltpu.VMEM((_CHUNK // 4, 4 * _EMBED), f32),
            pltpu.VMEM((_CHUNK, _EMBED), f32),
            pltpu.VMEM((_CHUNK // 4, 4 * _EMBED), f32),
            pltpu.SemaphoreType.DMA,
            pltpu.SemaphoreType.DMA,
            pltpu.SemaphoreType.DMA,
            pltpu.SemaphoreType.DMA,
        ],
    )
    def sc_kernel(idx_hbm, tab_hbm, out_hbm, idx_n, idx_v,
                  ea0, ob0, ea1, ob1, sg0, sg1, so0, so1):
        wid = lax.axis_index("s") * _NC + lax.axis_index("c")
        chunk0 = wid * _CHUNKS_PER_W
        bufs = ((ea0, ob0, sg0, so0), (ea1, ob1, sg1, so1))

        pltpu.sync_copy(idx_hbm.at[:, :, pl.ds(wid * 32, 32)], idx_n)
        iota16 = lax.iota(jnp.int32, _L)

        def stage(v, carry):
            il = jnp.full((_L,), (v >> 1) & 31, jnp.int32)
            id_ = iota16 + (v & 1) * _L
            ib = jnp.full((_L,), v >> 6, jnp.int32)
            vals = plsc.load_gather(idx_n, [il, id_, ib])
            idx_v[v >> 3, pl.ds((v & 7) * _L, _L)] = plsc.bitcast(
                vals, jnp.int32)
            return carry

        lax.fori_loop(0, 2048, stage, 0)

        for b in (0, 1):
            ea, _, sg, _ = bufs[b]
            pltpu.async_copy(tab_hbm.at[idx_v.at[b]], ea, sg)

        n_iter = _CHUNKS_PER_W // 2

        def body(j, carry):
            for b in (0, 1):
                ea, ob, sg, so = bufs[b]
                c = j * 2 + b
                pltpu.make_async_copy(tab_hbm.at[pl.ds(0, _CHUNK)], ea, sg).wait()

                @pl.when(j > 0)
                def _():
                    pltpu.make_async_copy(
                        ob, out_hbm.at[pl.ds(0, _CHUNK // 4)], so).wait()

                _compute_chunk(ea, ob)

                prow0 = (chunk0 + c) * (_CHUNK // 4)
                pltpu.async_copy(ob, out_hbm.at[pl.ds(prow0, _CHUNK // 4)], so)

                @pl.when(j < n_iter - 1)
                def _():
                    pltpu.async_copy(tab_hbm.at[idx_v.at[c + 2]], ea, sg)
            return carry

        lax.fori_loop(0, n_iter, body, 0)

        for b in (0, 1):
            _, ob, _, so = bufs[b]
            pltpu.make_async_copy(
                ob, out_hbm.at[pl.ds(0, _CHUNK // 4)], so).wait()

    return sc_kernel


_sc_kernel = _make_sc_kernel()


def kernel(x, emb_weight, pos_weight):
    b, l, d = x.shape
    e = emb_weight.shape[1]
    tab = emb_weight + pos_weight
    xt = lax.bitcast_convert_type(jnp.transpose(x, (1, 2, 0)), jnp.float32)
    out = _sc_kernel(xt, tab)
    return out.reshape(b, l, d, e)
